# Initial kernel scaffold; baseline (speedup 1.0000x reference)
import jax
import jax.numpy as jnp
from jax.experimental import pallas as pl


def kernel(x, edge_index, edge_attr, batch_index, W1, b1, W2, b2):
    raise NotImplementedError("write your pallas kernel here")


# speedup vs baseline: 2.3992x; 2.3992x over previous
import functools

import jax
import jax.numpy as jnp
from jax import lax
from jax.experimental import pallas as pl
from jax.experimental.pallas import tpu as pltpu
from jax.experimental.pallas import tpu_sc as plsc

N = 10000
D = 128
G = 64

NC = 2
NS = 16
NW = NC * NS
L = 16

RB = 16
NB = N // RB
BPW = (NB + NW - 1) // NW


def _sc_partial_sums(x, batch_index):
    mesh = plsc.VectorSubcoreMesh(
        core_axis_name="c", subcore_axis_name="s", num_cores=NC, num_subcores=NS
    )

    @functools.partial(
        pl.kernel,
        mesh=mesh,
        out_type=(
            jax.ShapeDtypeStruct((NW, G, D), jnp.float32),
            jax.ShapeDtypeStruct((NW, G, L), jnp.float32),
        ),
        scratch_types=[
            pltpu.VMEM((RB, D), jnp.float32),
            pltpu.VMEM((RB,), jnp.int32),
            pltpu.VMEM((G, D), jnp.float32),
            pltpu.VMEM((G, L), jnp.float32),
        ],
    )
    def k(x_hbm, bi_hbm, sums_out, cnts_out, xb, segb, acc, cacc):
        wid = lax.axis_index("s") * NC + lax.axis_index("c")
        zeros = jnp.zeros((L,), jnp.float32)

        def zero_row(i, carry):
            for j in range(D // L):
                acc[i, pl.ds(j * L, L)] = zeros
            cacc[i, :] = zeros
            return carry

        lax.fori_loop(0, G, zero_row, 0)

        def block_body(i, carry):
            blk = wid + NW * i

            @pl.when(blk < NB)
            def _():
                pltpu.sync_copy(x_hbm.at[pl.ds(blk * RB, RB)], xb)
                pltpu.sync_copy(bi_hbm.at[pl.ds(blk * RB, RB)], segb)
                e0 = (1 - jnp.minimum(lax.iota(jnp.int32, L), 1)).astype(
                    jnp.float32
                )
                segv = segb[...]
                for r in range(RB):
                    s = segv[r]
                    for j in range(D // L):
                        plsc.addupdate(
                            acc.at[s, pl.ds(j * L, L)], xb[r, pl.ds(j * L, L)]
                        )
                    plsc.addupdate(cacc.at[s], e0)

            return carry

        lax.fori_loop(0, BPW, block_body, 0)

        pltpu.sync_copy(acc, sums_out.at[wid])
        pltpu.sync_copy(cacc, cnts_out.at[wid])

    return k(x, batch_index)


def _tc_head(psums, pcnts, W1, b1, W2, b2):
    def body(ps_ref, pc_ref, w1_ref, b1_ref, w2_ref, b2_ref, o_ref):
        sums = jnp.sum(ps_ref[...], axis=0)
        cnt = jnp.sum(pc_ref[...], axis=(0, 2))
        pooled = sums / jnp.maximum(cnt, 1.0)[:, None]
        h = jnp.dot(pooled, w1_ref[...], preferred_element_type=jnp.float32)
        h = jnp.maximum(h + b1_ref[...], 0.0)
        h = jnp.dot(h, w2_ref[...], preferred_element_type=jnp.float32)
        o_ref[...] = jnp.maximum(h + b2_ref[...], 0.0)

    return pl.pallas_call(
        body,
        out_shape=jax.ShapeDtypeStruct((G, D), jnp.float32),
    )(psums, pcnts, W1, b1.reshape(1, D), W2, b2.reshape(1, D))


def kernel(x, edge_index, edge_attr, batch_index, W1, b1, W2, b2):
    del edge_index, edge_attr
    psums, pcnts = _sc_partial_sums(x, batch_index.astype(jnp.int32))
    return _tc_head(psums, pcnts, W1, b1, W2, b2)


# speedup vs baseline: 3.6696x; 1.5295x over previous
import functools

import jax
import jax.numpy as jnp
from jax import lax
from jax.experimental import pallas as pl
from jax.experimental.pallas import tpu as pltpu
from jax.experimental.pallas import tpu_sc as plsc

N = 10000
D = 128
G = 64

NC = 2
NS = 16
NW = NC * NS
L = 16

RB = 16
RPW = 320
NBW = RPW // RB


def _sc_partial_sums(x, batch_index):
    mesh = plsc.VectorSubcoreMesh(
        core_axis_name="c", subcore_axis_name="s", num_cores=NC, num_subcores=NS
    )

    @functools.partial(
        pl.kernel,
        mesh=mesh,
        out_type=(
            jax.ShapeDtypeStruct((NW, G, D), jnp.float32),
            jax.ShapeDtypeStruct((NW, G, L), jnp.float32),
        ),
        scratch_types=[
            pltpu.VMEM((RPW, D), jnp.float32),
            pltpu.VMEM((RPW,), jnp.int32),
            pltpu.VMEM((G, D), jnp.float32),
            pltpu.VMEM((G, L), jnp.float32),
        ],
    )
    def k(x_hbm, bi_hbm, sums_out, cnts_out, xb, segb, acc, cacc):
        wid = lax.axis_index("s") * NC + lax.axis_index("c")
        base = jnp.minimum(wid * RPW, N - RPW)
        lo = (wid * RPW - base) // RB

        pltpu.sync_copy(x_hbm.at[pl.ds(base, RPW)], xb)
        pltpu.sync_copy(bi_hbm.at[pl.ds(base, RPW)], segb)

        zeros = jnp.zeros((L,), jnp.float32)

        def zero_row(i, carry):
            for j in range(D // L):
                acc[i, pl.ds(j * L, L)] = zeros
            cacc[i, :] = zeros
            return carry

        lax.fori_loop(0, G, zero_row, 0)

        e0 = (1 - jnp.minimum(lax.iota(jnp.int32, L), 1)).astype(jnp.float32)

        def block_body(i, carry):
            segv = segb[pl.ds(i * RB, RB)]
            for r in range(RB):
                s = segv[r]
                row = i * RB + r
                for j in range(D // L):
                    plsc.addupdate(
                        acc.at[s, pl.ds(j * L, L)], xb[row, pl.ds(j * L, L)]
                    )
                plsc.addupdate(cacc.at[s], e0)
            return carry

        lax.fori_loop(lo, NBW, block_body, 0)

        pltpu.sync_copy(acc, sums_out.at[wid])
        pltpu.sync_copy(cacc, cnts_out.at[wid])

    return k(x, batch_index)


def _tc_head(psums, pcnts, W1, b1, W2, b2):
    def body(ps_ref, pc_ref, w1_ref, b1_ref, w2_ref, b2_ref, o_ref):
        sums = jnp.sum(ps_ref[...], axis=0)
        cnt = jnp.sum(pc_ref[...], axis=(0, 2))
        pooled = sums / jnp.maximum(cnt, 1.0)[:, None]
        h = jnp.dot(pooled, w1_ref[...], preferred_element_type=jnp.float32)
        h = jnp.maximum(h + b1_ref[...], 0.0)
        h = jnp.dot(h, w2_ref[...], preferred_element_type=jnp.float32)
        o_ref[...] = jnp.maximum(h + b2_ref[...], 0.0)

    return pl.pallas_call(
        body,
        out_shape=jax.ShapeDtypeStruct((G, D), jnp.float32),
    )(psums, pcnts, W1, b1.reshape(1, D), W2, b2.reshape(1, D))


def kernel(x, edge_index, edge_attr, batch_index, W1, b1, W2, b2):
    del edge_index, edge_attr
    psums, pcnts = _sc_partial_sums(x, batch_index.astype(jnp.int32))
    return _tc_head(psums, pcnts, W1, b1, W2, b2)


# speedup vs baseline: 4.5752x; 1.2468x over previous
import functools

import jax
import jax.numpy as jnp
from jax import lax
from jax.experimental import pallas as pl
from jax.experimental.pallas import tpu as pltpu
from jax.experimental.pallas import tpu_sc as plsc

N = 10000
D = 128
G = 64

NC = 2
NS = 16
NW = NC * NS
L = 16

CH = 80
NCH = 4
RPW = CH * NCH
GP = G + 1
OVL = NW * RPW - N


def _sc_partial_sums(x, bi80, zrows):
    mesh = plsc.VectorSubcoreMesh(
        core_axis_name="c", subcore_axis_name="s", num_cores=NC, num_subcores=NS
    )

    @functools.partial(
        pl.kernel,
        mesh=mesh,
        out_type=jax.ShapeDtypeStruct((NC, G, D), jnp.float32),
        scratch_types=[
            pltpu.VMEM((NCH, CH, D), jnp.float32),
            pltpu.VMEM((NCH, CH), jnp.int32),
            pltpu.VMEM((GP, D), jnp.float32),
            pltpu.VMEM_SHARED((GP, D), jnp.float32),
            pltpu.SemaphoreType.DMA,
            pltpu.SemaphoreType.DMA,
            pltpu.SemaphoreType.DMA,
            pltpu.SemaphoreType.DMA,
            pltpu.SemaphoreType.DMA,
        ],
    )
    def k(x_hbm, bi_hbm, z_hbm, sums_out, xb, segb, zb, accs, g0, g1, g2, g3, ssem):
        cid = lax.axis_index("c")
        sid = lax.axis_index("s")
        wid = sid * NC + cid
        base = jnp.minimum(wid * RPW, N - RPW)
        gsems = [g0, g1, g2, g3]

        gathers = []
        for j in range(NCH):
            off = pl.multiple_of(base + j * CH, CH)
            cp = pltpu.async_copy(x_hbm.at[pl.ds(off, CH)], xb.at[j], gsems[j])
            gathers.append(cp)
        pltpu.sync_copy(bi_hbm.at[wid], segb)

        @pl.when(sid == 0)
        def _():
            pltpu.sync_copy(z_hbm, zb)
            pltpu.sync_copy(zb, accs)

        plsc.subcore_barrier()

        scatters = []
        for j in range(NCH):
            gathers[j].wait()
            cp = pltpu.async_copy(xb.at[j], accs.at[segb.at[j]], ssem, add=True)
            scatters.append(cp)
        for cp in scatters:
            cp.wait()

        plsc.subcore_barrier()

        @pl.when(sid == 0)
        def _():
            pltpu.sync_copy(accs.at[pl.ds(0, G)], sums_out.at[cid])

    return k(x, bi80, zrows)


def _tc_head(psums, bi_pad, W1, b1, W2, b2):
    def body(ps_ref, bi_ref, w1_ref, b1_ref, w2_ref, b2_ref, o_ref):
        sums = ps_ref[0] + ps_ref[1]
        bi = bi_ref[...]
        ids = lax.broadcasted_iota(jnp.int32, (G, 1, 1), 0)
        cnt = jnp.sum((bi[None] == ids).astype(jnp.float32), axis=(1, 2))
        pooled = sums / jnp.maximum(cnt, 1.0)[:, None]
        h = jnp.dot(pooled, w1_ref[...], preferred_element_type=jnp.float32)
        h = jnp.maximum(h + b1_ref[...], 0.0)
        h = jnp.dot(h, w2_ref[...], preferred_element_type=jnp.float32)
        o_ref[...] = jnp.maximum(h + b2_ref[...], 0.0)

    return pl.pallas_call(
        body,
        out_shape=jax.ShapeDtypeStruct((G, D), jnp.float32),
    )(psums, bi_pad, W1, b1.reshape(1, D), W2, b2.reshape(1, D))


def kernel(x, edge_index, edge_attr, batch_index, W1, b1, W2, b2):
    del edge_index, edge_attr
    bi = batch_index.astype(jnp.int32)
    last_row = jnp.concatenate(
        [jnp.full((OVL,), G, jnp.int32), bi[N - (RPW - OVL):]]
    )
    bi_sc = jnp.concatenate([bi[: N - (RPW - OVL)], last_row]).reshape(
        NW, NCH, CH
    )
    zrows = jnp.zeros((GP, D), jnp.float32)
    psums = _sc_partial_sums(x, bi_sc, zrows)
    bi_pad = jnp.concatenate([bi, jnp.full((240,), G, jnp.int32)]).reshape(80, 128)
    return _tc_head(psums, bi_pad, W1, b1, W2, b2)
